# baseline (speedup 1.0000x reference)
import functools

import jax
import jax.numpy as jnp
from jax import lax
from jax.experimental import pallas as pl
from jax.experimental.pallas import tpu as pltpu
from jax.experimental.pallas import tpu_sc as plsc

U_ROWS = 5000
I_ROWS = 5000
N_ROWS = U_ROWS + I_ROWS
D = 16
NC = 2
NS = 16
CHUNK = 128
NBUF = 8
NCH = 160


_GATHER_DN = lax.GatherDimensionNumbers(
    offset_dims=(), collapsed_slice_dims=(0,), start_index_map=(0,))


def _lane_bcast(v, k):
    idx = jnp.full((16, 1), k, jnp.int32)
    return lax.gather(v, idx, _GATHER_DN, (1,),
                      mode=lax.GatherScatterMode.PROMISE_IN_BOUNDS)


def _spmm_body(tab_ref, src_ref, dst_ref, val_ref, out_ref,
               src_v, dst_v, val_v, rows_v, msg_v, zeros_v, out_sh,
               gsem, ssem):
    cid = lax.axis_index("c")
    sid = lax.axis_index("s")
    rpt = N_ROWS // NS

    @pl.when(cid == 0)
    def _():
        def zero_body(i, c):
            zeros_v[i, :] = jnp.zeros((D,), jnp.float32)
            return c
        lax.fori_loop(0, rpt, zero_body, 0)
        pltpu.sync_copy(zeros_v, out_sh.at[pl.ds(sid * rpt, rpt)])

        base = sid * NCH
        pltpu.sync_copy(src_ref.at[pl.ds(base, NCH)], src_v)
        pltpu.sync_copy(dst_ref.at[pl.ds(base, NCH)], dst_v)
        pltpu.sync_copy(val_ref.at[pl.ds(base, NCH)], val_v)
        plsc.subcore_barrier()

        for b in range(NBUF):
            pltpu.async_copy(tab_ref.at[src_v.at[b]], rows_v.at[b],
                             gsem.at[b])

        def super_body(jj, c):
            for b in range(NBUF):
                j = jj * NBUF + b
                pltpu.make_async_copy(tab_ref.at[src_v.at[j]], rows_v.at[b],
                                      gsem.at[b]).wait()

                @pl.when(jj > 0)
                def _():
                    pltpu.make_async_copy(msg_v.at[b],
                                          out_sh.at[dst_v.at[j]],
                                          ssem.at[b]).wait()

                def scale_group(g, c2):
                    gb = g * 16
                    vblk = val_v[j, pl.ds(gb, 16)]
                    for k in range(16):
                        ee = gb + k
                        vb = _lane_bcast(vblk, k)
                        msg_v[b, ee, :] = rows_v[b, ee, :] * vb
                    return c2
                lax.fori_loop(0, CHUNK // 16, scale_group, 0)

                @pl.when(j + NBUF < NCH)
                def _():
                    pltpu.async_copy(tab_ref.at[src_v.at[j + NBUF]],
                                     rows_v.at[b], gsem.at[b])

                pltpu.async_copy(msg_v.at[b], out_sh.at[dst_v.at[j]],
                                 ssem.at[b], add=True)
            return c
        lax.fori_loop(0, NCH // NBUF, super_body, 0)

        for b in range(NBUF):
            pltpu.make_async_copy(msg_v.at[b], out_sh.at[dst_v.at[b]],
                                  ssem.at[b]).wait()
        plsc.subcore_barrier()

        pltpu.sync_copy(out_sh.at[pl.ds(sid * rpt, rpt)],
                        out_ref.at[pl.ds(sid * rpt, rpt)])


def _make_spmm():
    mesh = plsc.VectorSubcoreMesh(core_axis_name="c", subcore_axis_name="s",
                                  num_cores=NC, num_subcores=NS)
    return pl.kernel(
        _spmm_body,
        out_type=jax.ShapeDtypeStruct((N_ROWS, D), jnp.float32),
        mesh=mesh,
        compiler_params=pltpu.CompilerParams(use_tc_tiling_on_sc=False),
        scratch_types=[
            pltpu.VMEM((NCH, CHUNK), jnp.int32),
            pltpu.VMEM((NCH, CHUNK), jnp.int32),
            pltpu.VMEM((NCH, CHUNK), jnp.float32),
            pltpu.VMEM((NBUF, CHUNK, D), jnp.float32),
            pltpu.VMEM((NBUF, CHUNK, D), jnp.float32),
            pltpu.VMEM((N_ROWS // NS, D), jnp.float32),
            pltpu.VMEM_SHARED((N_ROWS, D), jnp.float32),
            pltpu.SemaphoreType.DMA((NBUF,)),
            pltpu.SemaphoreType.DMA((NBUF,)),
        ],
    )


def _mean_body(x0_ref, t1_ref, t2_ref, t3_ref, u_ref, i_ref):
    s = x0_ref[...] + t1_ref[...] + t2_ref[...] + t3_ref[...]
    light = s * 0.25
    u_ref[...] = light[:U_ROWS]
    i_ref[...] = light[U_ROWS:]


LOG2E = 1.4426950408889634
LN2 = 0.6931471805599453


def _loss_body(u_ref, it_ref, lab_ref, loss_ref):
    i = pl.program_id(0)
    u = u_ref[...]
    it = it_ref[...]
    s2 = lax.dot_general(u * LOG2E, it, (((1,), (1,)), ((), ())),
                         preferred_element_type=jnp.float32)
    t = jnp.exp2(-jnp.abs(s2))
    soft = jnp.maximum(s2, 0.0) + jnp.log2(1.0 + t)
    part = LN2 * jnp.sum(soft - s2 * lab_ref[...])

    @pl.when(i == 0)
    def _():
        loss_ref[...] = jnp.zeros_like(loss_ref)

    loss_ref[...] = loss_ref[...] + part

    @pl.when(i == pl.num_programs(0) - 1)
    def _():
        loss_ref[...] = loss_ref[...] * (1.0 / (U_ROWS * I_ROWS))


BU = 200


def kernel(users_emb, items_emb, adj_indices, adj_values, labels):
    e = adj_values.shape[0]
    total_chunks = NS * NCH
    e_pad = total_chunks * CHUNK
    assert e <= e_pad

    x0 = jnp.concatenate([users_emb, items_emb], axis=0)
    dst = jnp.pad(adj_indices[0], (0, e_pad - e)).reshape(total_chunks, CHUNK)
    src = jnp.pad(adj_indices[1], (0, e_pad - e)).reshape(total_chunks, CHUNK)
    val = jnp.pad(adj_values, (0, e_pad - e)).reshape(total_chunks, CHUNK)

    spmm = _make_spmm()
    t1 = spmm(x0, src, dst, val)
    t2 = spmm(t1, src, dst, val)
    t3 = spmm(t2, src, dst, val)

    user_all, item_all = pl.pallas_call(
        _mean_body,
        out_shape=[jax.ShapeDtypeStruct((U_ROWS, D), jnp.float32),
                   jax.ShapeDtypeStruct((I_ROWS, D), jnp.float32)],
    )(x0, t1, t2, t3)

    loss = pl.pallas_call(
        _loss_body,
        grid=(U_ROWS // BU,),
        in_specs=[
            pl.BlockSpec((BU, D), lambda i: (i, 0)),
            pl.BlockSpec((I_ROWS, D), lambda i: (0, 0)),
            pl.BlockSpec((BU, I_ROWS), lambda i: (i, 0)),
        ],
        out_specs=pl.BlockSpec((1, 1), lambda i: (0, 0)),
        out_shape=jax.ShapeDtypeStruct((1, 1), jnp.float32),
    )(user_all, item_all, labels)

    return (loss[0, 0], user_all, item_all)


# speedup vs baseline: 1.1658x; 1.1658x over previous
import functools

import jax
import jax.numpy as jnp
from jax import lax
from jax.experimental import pallas as pl
from jax.experimental.pallas import tpu as pltpu
from jax.experimental.pallas import tpu_sc as plsc

U_ROWS = 5000
I_ROWS = 5000
N_ROWS = U_ROWS + I_ROWS
D = 16
NC = 2
NS = 16
CHUNK = 128
NBUF = 8
NCH = 160


_GATHER_DN = lax.GatherDimensionNumbers(
    offset_dims=(), collapsed_slice_dims=(0,), start_index_map=(0,))


def _lane_bcast(v, k):
    idx = jnp.full((16, 1), k, jnp.int32)
    return lax.gather(v, idx, _GATHER_DN, (1,),
                      mode=lax.GatherScatterMode.PROMISE_IN_BOUNDS)


def _spmm_body(tab_ref, src_ref, dst_ref, val_ref, out_ref,
               src_v, dst_v, val_v, rows_v, msg_v, zeros_v, tab_sh, out_sh,
               gsem, ssem):
    cid = lax.axis_index("c")
    sid = lax.axis_index("s")
    rpt = N_ROWS // NS

    @pl.when(cid == 0)
    def _():
        def zero_body(i, c):
            zeros_v[i, :] = jnp.zeros((D,), jnp.float32)
            return c
        lax.fori_loop(0, rpt, zero_body, 0)
        pltpu.sync_copy(zeros_v, out_sh.at[pl.ds(sid * rpt, rpt)])

        pltpu.sync_copy(tab_ref.at[pl.ds(sid * rpt, rpt)],
                        tab_sh.at[pl.ds(sid * rpt, rpt)])

        base = sid * NCH
        pltpu.sync_copy(src_ref.at[pl.ds(base, NCH)], src_v)
        pltpu.sync_copy(dst_ref.at[pl.ds(base, NCH)], dst_v)
        pltpu.sync_copy(val_ref.at[pl.ds(base, NCH)], val_v)
        plsc.subcore_barrier()

        for b in range(NBUF):
            pltpu.async_copy(tab_sh.at[src_v.at[b]], rows_v.at[b],
                             gsem.at[b])

        def super_body(jj, c):
            for b in range(NBUF):
                j = jj * NBUF + b
                pltpu.make_async_copy(tab_sh.at[src_v.at[j]], rows_v.at[b],
                                      gsem.at[b]).wait()

                @pl.when(jj > 0)
                def _():
                    pltpu.make_async_copy(msg_v.at[b],
                                          out_sh.at[dst_v.at[j]],
                                          ssem.at[b]).wait()

                @plsc.parallel_loop(0, CHUNK // 16, unroll=2)
                def _(g):
                    gb = g * 16
                    vblk = val_v[j, pl.ds(gb, 16)]
                    for k in range(16):
                        ee = gb + k
                        vb = _lane_bcast(vblk, k)
                        msg_v[b, ee, :] = rows_v[b, ee, :] * vb

                @pl.when(j + NBUF < NCH)
                def _():
                    pltpu.async_copy(tab_sh.at[src_v.at[j + NBUF]],
                                     rows_v.at[b], gsem.at[b])

                pltpu.async_copy(msg_v.at[b], out_sh.at[dst_v.at[j]],
                                 ssem.at[b], add=True)
            return c
        lax.fori_loop(0, NCH // NBUF, super_body, 0)

        for b in range(NBUF):
            pltpu.make_async_copy(msg_v.at[b], out_sh.at[dst_v.at[b]],
                                  ssem.at[b]).wait()
        plsc.subcore_barrier()

        pltpu.sync_copy(out_sh.at[pl.ds(sid * rpt, rpt)],
                        out_ref.at[pl.ds(sid * rpt, rpt)])


def _make_spmm():
    mesh = plsc.VectorSubcoreMesh(core_axis_name="c", subcore_axis_name="s",
                                  num_cores=NC, num_subcores=NS)
    return pl.kernel(
        _spmm_body,
        out_type=jax.ShapeDtypeStruct((N_ROWS, D), jnp.float32),
        mesh=mesh,
        compiler_params=pltpu.CompilerParams(use_tc_tiling_on_sc=False),
        scratch_types=[
            pltpu.VMEM((NCH, CHUNK), jnp.int32),
            pltpu.VMEM((NCH, CHUNK), jnp.int32),
            pltpu.VMEM((NCH, CHUNK), jnp.float32),
            pltpu.VMEM((NBUF, CHUNK, D), jnp.float32),
            pltpu.VMEM((NBUF, CHUNK, D), jnp.float32),
            pltpu.VMEM((N_ROWS // NS, D), jnp.float32),
            pltpu.VMEM_SHARED((N_ROWS, D), jnp.float32),
            pltpu.VMEM_SHARED((N_ROWS, D), jnp.float32),
            pltpu.SemaphoreType.DMA((NBUF,)),
            pltpu.SemaphoreType.DMA((NBUF,)),
        ],
    )


def _mean_body(x0_ref, t1_ref, t2_ref, t3_ref, u_ref, i_ref):
    s = x0_ref[...] + t1_ref[...] + t2_ref[...] + t3_ref[...]
    light = s * 0.25
    u_ref[...] = light[:U_ROWS]
    i_ref[...] = light[U_ROWS:]


LOG2E = 1.4426950408889634
LN2 = 0.6931471805599453


def _loss_body(u_ref, it_ref, lab_ref, loss_ref):
    i = pl.program_id(0)
    u = u_ref[...]
    it = it_ref[...]
    s2 = lax.dot_general(u * LOG2E, it, (((1,), (1,)), ((), ())),
                         preferred_element_type=jnp.float32)
    t = jnp.exp2(-jnp.abs(s2))
    soft = jnp.maximum(s2, 0.0) + jnp.log2(1.0 + t)
    part = LN2 * jnp.sum(soft - s2 * lab_ref[...])

    @pl.when(i == 0)
    def _():
        loss_ref[...] = jnp.zeros_like(loss_ref)

    loss_ref[...] = loss_ref[...] + part

    @pl.when(i == pl.num_programs(0) - 1)
    def _():
        loss_ref[...] = loss_ref[...] * (1.0 / (U_ROWS * I_ROWS))


BU = 200


def kernel(users_emb, items_emb, adj_indices, adj_values, labels):
    e = adj_values.shape[0]
    total_chunks = NS * NCH
    e_pad = total_chunks * CHUNK
    assert e <= e_pad

    x0 = jnp.concatenate([users_emb, items_emb], axis=0)
    dst = jnp.pad(adj_indices[0], (0, e_pad - e)).reshape(total_chunks, CHUNK)
    src = jnp.pad(adj_indices[1], (0, e_pad - e)).reshape(total_chunks, CHUNK)
    val = jnp.pad(adj_values, (0, e_pad - e)).reshape(total_chunks, CHUNK)

    spmm = _make_spmm()
    t1 = spmm(x0, src, dst, val)
    t2 = spmm(t1, src, dst, val)
    t3 = spmm(t2, src, dst, val)

    user_all, item_all = pl.pallas_call(
        _mean_body,
        out_shape=[jax.ShapeDtypeStruct((U_ROWS, D), jnp.float32),
                   jax.ShapeDtypeStruct((I_ROWS, D), jnp.float32)],
    )(x0, t1, t2, t3)

    loss = pl.pallas_call(
        _loss_body,
        grid=(U_ROWS // BU,),
        in_specs=[
            pl.BlockSpec((BU, D), lambda i: (i, 0)),
            pl.BlockSpec((I_ROWS, D), lambda i: (0, 0)),
            pl.BlockSpec((BU, I_ROWS), lambda i: (i, 0)),
        ],
        out_specs=pl.BlockSpec((1, 1), lambda i: (0, 0)),
        out_shape=jax.ShapeDtypeStruct((1, 1), jnp.float32),
    )(user_all, item_all, labels)

    return (loss[0, 0], user_all, item_all)


# speedup vs baseline: 1.2459x; 1.0687x over previous
import functools

import jax
import jax.numpy as jnp
from jax import lax
from jax.experimental import pallas as pl
from jax.experimental.pallas import tpu as pltpu
from jax.experimental.pallas import tpu_sc as plsc

U_ROWS = 5000
I_ROWS = 5000
N_ROWS = U_ROWS + I_ROWS
D = 16
NC = 2
NS = 16
CHUNK = 128
NBUF = 8
NCH = 160


_GATHER_DN = lax.GatherDimensionNumbers(
    offset_dims=(), collapsed_slice_dims=(0,), start_index_map=(0,))


def _lane_bcast(v, k):
    idx = jnp.full((16, 1), k, jnp.int32)
    return lax.gather(v, idx, _GATHER_DN, (1,),
                      mode=lax.GatherScatterMode.PROMISE_IN_BOUNDS)


def _gcn_body(x0_ref, src_ref, dst_ref, val_ref, t1_ref, t2_ref, t3_ref,
              src_v, dst_v, val_v, rows_v, msg_v, zeros_v, sh_a, sh_b,
              gsem, ssem):
    cid = lax.axis_index("c")
    sid = lax.axis_index("s")
    rpt = N_ROWS // NS
    sl = pl.ds(sid * rpt, rpt)

    @pl.when(cid == 0)
    def _():
        def zero_body(i, c):
            zeros_v[i, :] = jnp.zeros((D,), jnp.float32)
            return c
        lax.fori_loop(0, rpt, zero_body, 0)
        pltpu.sync_copy(zeros_v, sh_b.at[sl])
        pltpu.sync_copy(x0_ref.at[sl], sh_a.at[sl])
        base = sid * NCH
        pltpu.sync_copy(src_ref.at[pl.ds(base, NCH)], src_v)
        pltpu.sync_copy(dst_ref.at[pl.ds(base, NCH)], dst_v)
        pltpu.sync_copy(val_ref.at[pl.ds(base, NCH)], val_v)
        plsc.subcore_barrier()

        def run_layer(tab_sh, acc_sh):
            for b in range(NBUF):
                pltpu.async_copy(tab_sh.at[src_v.at[b]], rows_v.at[b],
                                 gsem.at[b])

            def super_body(jj, c):
                for b in range(NBUF):
                    j = jj * NBUF + b
                    pltpu.make_async_copy(tab_sh.at[src_v.at[j]],
                                          rows_v.at[b], gsem.at[b]).wait()

                    @pl.when(jj > 0)
                    def _():
                        pltpu.make_async_copy(msg_v.at[b],
                                              acc_sh.at[dst_v.at[j]],
                                              ssem.at[b]).wait()

                    @plsc.parallel_loop(0, CHUNK // 16, unroll=2)
                    def _(g):
                        gb = g * 16
                        vblk = val_v[j, pl.ds(gb, 16)]
                        for k in range(16):
                            ee = gb + k
                            vb = _lane_bcast(vblk, k)
                            msg_v[b, ee, :] = rows_v[b, ee, :] * vb

                    @pl.when(j + NBUF < NCH)
                    def _():
                        pltpu.async_copy(tab_sh.at[src_v.at[j + NBUF]],
                                         rows_v.at[b], gsem.at[b])

                    pltpu.async_copy(msg_v.at[b], acc_sh.at[dst_v.at[j]],
                                     ssem.at[b], add=True)
                return c
            lax.fori_loop(0, NCH // NBUF, super_body, 0)

            for b in range(NBUF):
                pltpu.make_async_copy(msg_v.at[b], acc_sh.at[dst_v.at[b]],
                                      ssem.at[b]).wait()
            plsc.subcore_barrier()

        bufs = (sh_a, sh_b)
        touts = (t1_ref, t2_ref, t3_ref)
        for layer in range(3):
            tab_sh, acc_sh = bufs[layer % 2], bufs[(layer + 1) % 2]
            run_layer(tab_sh, acc_sh)
            pltpu.sync_copy(acc_sh.at[sl], touts[layer].at[sl])
            if layer < 2:
                pltpu.sync_copy(zeros_v, tab_sh.at[sl])
                plsc.subcore_barrier()


def _make_gcn():
    mesh = plsc.VectorSubcoreMesh(core_axis_name="c", subcore_axis_name="s",
                                  num_cores=NC, num_subcores=NS)
    t = jax.ShapeDtypeStruct((N_ROWS, D), jnp.float32)
    return pl.kernel(
        _gcn_body,
        out_type=[t, t, t],
        mesh=mesh,
        compiler_params=pltpu.CompilerParams(use_tc_tiling_on_sc=False),
        scratch_types=[
            pltpu.VMEM((NCH, CHUNK), jnp.int32),
            pltpu.VMEM((NCH, CHUNK), jnp.int32),
            pltpu.VMEM((NCH, CHUNK), jnp.float32),
            pltpu.VMEM((NBUF, CHUNK, D), jnp.float32),
            pltpu.VMEM((NBUF, CHUNK, D), jnp.float32),
            pltpu.VMEM((N_ROWS // NS, D), jnp.float32),
            pltpu.VMEM_SHARED((N_ROWS, D), jnp.float32),
            pltpu.VMEM_SHARED((N_ROWS, D), jnp.float32),
            pltpu.SemaphoreType.DMA((NBUF,)),
            pltpu.SemaphoreType.DMA((NBUF,)),
        ],
    )


def _mean_body(x0_ref, t1_ref, t2_ref, t3_ref, u_ref, i_ref):
    s = x0_ref[...] + t1_ref[...] + t2_ref[...] + t3_ref[...]
    light = s * 0.25
    u_ref[...] = light[:U_ROWS]
    i_ref[...] = light[U_ROWS:]


LOG2E = 1.4426950408889634
LN2 = 0.6931471805599453


def _loss_body(u_ref, it_ref, lab_ref, loss_ref):
    i = pl.program_id(0)
    u = u_ref[...]
    it = it_ref[...]
    s2 = lax.dot_general(u * LOG2E, it, (((1,), (1,)), ((), ())),
                         preferred_element_type=jnp.float32)
    t = jnp.exp2(-jnp.abs(s2))
    soft = jnp.maximum(s2, 0.0) + jnp.log2(1.0 + t)
    part = LN2 * jnp.sum(soft - s2 * lab_ref[...])

    @pl.when(i == 0)
    def _():
        loss_ref[...] = jnp.zeros_like(loss_ref)

    loss_ref[...] = loss_ref[...] + part

    @pl.when(i == pl.num_programs(0) - 1)
    def _():
        loss_ref[...] = loss_ref[...] * (1.0 / (U_ROWS * I_ROWS))


BU = 200


def kernel(users_emb, items_emb, adj_indices, adj_values, labels):
    e = adj_values.shape[0]
    total_chunks = NS * NCH
    e_pad = total_chunks * CHUNK
    assert e <= e_pad

    x0 = jnp.concatenate([users_emb, items_emb], axis=0)
    dst = jnp.pad(adj_indices[0], (0, e_pad - e)).reshape(total_chunks, CHUNK)
    src = jnp.pad(adj_indices[1], (0, e_pad - e)).reshape(total_chunks, CHUNK)
    val = jnp.pad(adj_values, (0, e_pad - e)).reshape(total_chunks, CHUNK)

    t1, t2, t3 = _make_gcn()(x0, src, dst, val)

    user_all, item_all = pl.pallas_call(
        _mean_body,
        out_shape=[jax.ShapeDtypeStruct((U_ROWS, D), jnp.float32),
                   jax.ShapeDtypeStruct((I_ROWS, D), jnp.float32)],
    )(x0, t1, t2, t3)

    loss = pl.pallas_call(
        _loss_body,
        grid=(U_ROWS // BU,),
        in_specs=[
            pl.BlockSpec((BU, D), lambda i: (i, 0)),
            pl.BlockSpec((I_ROWS, D), lambda i: (0, 0)),
            pl.BlockSpec((BU, I_ROWS), lambda i: (i, 0)),
        ],
        out_specs=pl.BlockSpec((1, 1), lambda i: (0, 0)),
        out_shape=jax.ShapeDtypeStruct((1, 1), jnp.float32),
    )(user_all, item_all, labels)

    return (loss[0, 0], user_all, item_all)
